# baseline (speedup 1.0000x reference)
import functools

import jax
import jax.numpy as jnp
from jax import lax
from jax.experimental import pallas as pl
from jax.experimental.pallas import tpu as pltpu
from jax.experimental.pallas import tpu_sc as plsc

VOCAB, MAXLEN, EMBED, BATCH = 100000, 200, 32, 4096
NC, NS = 2, 16
NW = NC * NS
BW = BATCH // NW
TC = 5
N_CHUNKS = MAXLEN // TC
PAD = 133


def _build():
  mesh = plsc.VectorSubcoreMesh(core_axis_name="c", subcore_axis_name="s")

  @functools.partial(
      pl.kernel,
      out_type=jax.ShapeDtypeStruct((MAXLEN, EMBED // 8, BATCH // 128, 8, 128),
                                    jnp.float32),
      mesh=mesh,
      compiler_params=pltpu.CompilerParams(use_tc_tiling_on_sc=False,
                                           needs_layout_passes=False),
      scratch_types=[
          pltpu.VMEM((2, TC, BW), jnp.int32),
          pltpu.VMEM((MAXLEN, EMBED), jnp.float32),
          pltpu.VMEM((2, TC, BW, EMBED), jnp.float32),
          pltpu.VMEM((2, TC * 4 * 8, PAD), jnp.float32),
          pltpu.SemaphoreType.DMA,
          pltpu.SemaphoreType.DMA,
          pltpu.SemaphoreType.DMA,
          pltpu.SemaphoreType.DMA,
          pltpu.SemaphoreType.DMA,
          pltpu.SemaphoreType.DMA,
      ],
  )
  def emb_kernel(tok_hbm, xt_hbm, pos_hbm, out_hbm,
                 idx_v, pos_v, rows_v, st_v, g0, g1, s0, s1, i0, i1):
    gsem = (g0, g1)
    ssem = (s0, s1)
    isem = (i0, i1)
    wid = lax.axis_index("s") * NC + lax.axis_index("c")
    bbase = wid * BW

    pltpu.sync_copy(pos_hbm, pos_v)

    lane = lax.iota(jnp.int32, 16)
    i_eg0 = lane // 8
    i_eg1 = i_eg0 + 2
    i_el = lane % 8

    def idx_desc(k, b):
      return pltpu.make_async_copy(
          xt_hbm.at[pl.ds(k * TC, TC), pl.ds(bbase, BW)],
          idx_v.at[b], isem[b])

    def gather_descs(b):
      return [pltpu.make_async_copy(
                  tok_hbm.at[idx_v.at[b, ti]], rows_v.at[b, ti], gsem[b])
              for ti in range(TC)]

    def store_descs(k, b):
      return [pltpu.make_async_copy(
                  st_v.at[b, pl.ds((ti * 4 + eg) * 8, 8), pl.ds(0, 128)],
                  out_hbm.at[k * TC + ti, eg, wid], ssem[b])
              for ti in range(TC) for eg in range(4)]

    idx_desc(0, 0).start()
    idx_desc(0, 0).wait()
    for d in gather_descs(0):
      d.start()
    idx_desc(1, 1).start()

    @pl.loop(0, N_CHUNKS, step=2)
    def _chunks(k0):
      for b in range(2):
        k = k0 + b
        nb = 1 - b

        for d in gather_descs(b):
          d.wait()

        @pl.when(k + 2 < N_CHUNKS)
        def _prefetch_idx():
          idx_desc(k + 2, b).start()

        @pl.when(k + 1 < N_CHUNKS)
        def _issue_next_gather():
          idx_desc(k + 1, nb).wait()
          for d in gather_descs(nb):
            d.start()

        @pl.when(k >= 2)
        def _drain_store():
          for d in store_descs(k - 2, b):
            d.wait()

        st_b = st_v.at[b]

        for ti in range(TC):
          t = k * TC + ti
          p0 = pos_v[t, pl.ds(0, 16)]
          p1 = pos_v[t, pl.ds(16, 16)]
          i_row0 = ti * 32 + lane
          i_row1 = i_row0 + 16

          @pl.loop(0, BW, step=16)
          def _rows(bl0):
            for d in range(16):
              bl = bl0 + d
              i_bl = jnp.full((16,), 0, jnp.int32) + bl
              r0 = rows_v[b, ti, bl, pl.ds(0, 16)]
              r1 = rows_v[b, ti, bl, pl.ds(16, 16)]
              plsc.store_scatter(st_b, [i_row0, i_bl], r0 + p0)
              plsc.store_scatter(st_b, [i_row1, i_bl], r1 + p1)

        for d in store_descs(k, b):
          d.start()

    for d in store_descs(N_CHUNKS - 2, 0):
      d.wait()
    for d in store_descs(N_CHUNKS - 1, 1):
      d.wait()

  return emb_kernel


_emb = _build()


def kernel(x, token_table, pos_table):
  xt = x.T.astype(jnp.int32)
  out5 = _emb(token_table, xt, pos_table)
  return out5.transpose(2, 4, 0, 1, 3).reshape(BATCH, MAXLEN, EMBED)


# speedup vs baseline: 1.9275x; 1.9275x over previous
import functools

import jax
import jax.numpy as jnp
from jax import lax
from jax.experimental import pallas as pl
from jax.experimental.pallas import tpu as pltpu
from jax.experimental.pallas import tpu_sc as plsc

VOCAB, MAXLEN, EMBED, BATCH = 100000, 200, 32, 4096
NC, NS = 2, 16
NW = NC * NS
BW = BATCH // NW
TC = 5
N_CHUNKS = MAXLEN // TC
PAD = 133


def _build():
  mesh = plsc.VectorSubcoreMesh(core_axis_name="c", subcore_axis_name="s")

  @functools.partial(
      pl.kernel,
      out_type=jax.ShapeDtypeStruct((MAXLEN, EMBED // 8, BATCH // 128, 8, 128),
                                    jnp.float32),
      mesh=mesh,
      compiler_params=pltpu.CompilerParams(use_tc_tiling_on_sc=False,
                                           needs_layout_passes=False),
      scratch_types=[
          pltpu.VMEM((2, TC, BW), jnp.int32),
          pltpu.VMEM((MAXLEN, EMBED), jnp.float32),
          pltpu.VMEM((2, TC, BW, EMBED), jnp.float32),
          pltpu.VMEM((2, TC * 4 * 8, PAD), jnp.float32),
          pltpu.SemaphoreType.DMA,
          pltpu.SemaphoreType.DMA,
          pltpu.SemaphoreType.DMA,
          pltpu.SemaphoreType.DMA,
          pltpu.SemaphoreType.DMA,
          pltpu.SemaphoreType.DMA,
      ],
  )
  def emb_kernel(tok_hbm, xt_hbm, pos_hbm, out_hbm,
                 idx_v, pos_v, rows_v, st_v, g0, g1, s0, s1, i0, i1):
    gsem = (g0, g1)
    ssem = (s0, s1)
    isem = (i0, i1)
    wid = lax.axis_index("s") * NC + lax.axis_index("c")
    bbase = wid * BW

    pltpu.sync_copy(pos_hbm, pos_v)

    lane = lax.iota(jnp.int32, 16)
    i_eg0 = lane // 8
    i_eg1 = i_eg0 + 2
    i_el = lane % 8

    def idx_desc(k, b):
      return pltpu.make_async_copy(
          xt_hbm.at[pl.ds(k * TC, TC), pl.ds(bbase, BW)],
          idx_v.at[b], isem[b])

    def gather_descs(b):
      return [pltpu.make_async_copy(
                  tok_hbm.at[idx_v.at[b, ti]], rows_v.at[b, ti], gsem[b])
              for ti in range(TC)]

    def store_descs(k, b):
      return [pltpu.make_async_copy(
                  st_v.at[b, pl.ds((ti * 4 + eg) * 8, 8), pl.ds(0, 128)],
                  out_hbm.at[k * TC + ti, eg, wid], ssem[b])
              for ti in range(TC) for eg in range(4)]

    idx_desc(0, 0).start()
    idx_desc(0, 0).wait()
    for d in gather_descs(0):
      d.start()
    idx_desc(1, 1).start()

    @pl.loop(0, N_CHUNKS, step=2)
    def _chunks(k0):
      for b in range(2):
        k = k0 + b
        nb = 1 - b

        for d in gather_descs(b):
          d.wait()

        @pl.when(k + 2 < N_CHUNKS)
        def _prefetch_idx():
          idx_desc(k + 2, b).start()

        @pl.when(k + 1 < N_CHUNKS)
        def _issue_next_gather():
          idx_desc(k + 1, nb).wait()
          for d in gather_descs(nb):
            d.start()

        @pl.when(k >= 2)
        def _drain_store():
          for d in store_descs(k - 2, b):
            d.wait()

        st_b = st_v.at[b]

        for ti in range(TC):
          t = k * TC + ti
          p0 = pos_v[t, pl.ds(0, 16)]
          p1 = pos_v[t, pl.ds(16, 16)]
          i_row0 = ti * 32 + lane
          i_row1 = i_row0 + 16

          @plsc.parallel_loop(0, BW, 8)
          def _rows(bl0):
            i_bl0 = jnp.zeros((16,), jnp.int32) + bl0
            loads = []
            for d in range(8):
              bl = bl0 + d
              loads.append((rows_v[b, ti, bl, pl.ds(0, 16)],
                            rows_v[b, ti, bl, pl.ds(16, 16)]))
            ibls = [i_bl0 + d if d else i_bl0 for d in range(8)]
            sums = [(r0 + p0, r1 + p1) for r0, r1 in loads]
            for d in range(8):
              plsc.store_scatter(st_b, [i_row0, ibls[d]], sums[d][0])
              plsc.store_scatter(st_b, [i_row1, ibls[d]], sums[d][1])

        for d in store_descs(k, b):
          d.start()

    for d in store_descs(N_CHUNKS - 2, 0):
      d.wait()
    for d in store_descs(N_CHUNKS - 1, 1):
      d.wait()

  return emb_kernel


_emb = _build()


def kernel(x, token_table, pos_table):
  xt = x.T.astype(jnp.int32)
  out5 = _emb(token_table, xt, pos_table)
  return out5.transpose(2, 4, 0, 1, 3).reshape(BATCH, MAXLEN, EMBED)


# speedup vs baseline: 1.9465x; 1.0099x over previous
import functools

import jax
import jax.numpy as jnp
from jax import lax
from jax.experimental import pallas as pl
from jax.experimental.pallas import tpu as pltpu
from jax.experimental.pallas import tpu_sc as plsc

VOCAB, MAXLEN, EMBED, BATCH = 100000, 200, 32, 4096
NC, NS = 2, 16
NW = NC * NS
BW = BATCH // NW
TC = 5
N_CHUNKS = MAXLEN // TC
PAD = 133


def _build():
  mesh = plsc.VectorSubcoreMesh(core_axis_name="c", subcore_axis_name="s")

  @functools.partial(
      pl.kernel,
      out_type=jax.ShapeDtypeStruct((MAXLEN, EMBED // 8, BATCH // 128, 8, 128),
                                    jnp.float32),
      mesh=mesh,
      compiler_params=pltpu.CompilerParams(use_tc_tiling_on_sc=False,
                                           needs_layout_passes=False),
      scratch_types=[
          pltpu.VMEM((2, TC, BW), jnp.int32),
          pltpu.VMEM((MAXLEN, EMBED), jnp.float32),
          pltpu.VMEM((2, TC, BW, EMBED), jnp.float32),
          pltpu.VMEM((2, TC, 4, 8, PAD), jnp.float32),
          pltpu.SemaphoreType.DMA,
          pltpu.SemaphoreType.DMA,
          pltpu.SemaphoreType.DMA,
          pltpu.SemaphoreType.DMA,
          pltpu.SemaphoreType.DMA,
          pltpu.SemaphoreType.DMA,
      ],
  )
  def emb_kernel(tok_hbm, xt_hbm, pos_hbm, out_hbm,
                 idx_v, pos_v, rows_v, st_v, g0, g1, s0, s1, i0, i1):
    gsem = (g0, g1)
    ssem = (s0, s1)
    isem = (i0, i1)
    wid = lax.axis_index("s") * NC + lax.axis_index("c")
    bbase = wid * BW

    pltpu.sync_copy(pos_hbm, pos_v)

    lane = lax.iota(jnp.int32, 16)
    i_eg0 = lane // 8
    i_eg1 = i_eg0 + 2
    i_el = lane % 8

    def idx_desc(k, b):
      return pltpu.make_async_copy(
          xt_hbm.at[pl.ds(k * TC, TC), pl.ds(bbase, BW)],
          idx_v.at[b], isem[b])

    def gather_descs(b):
      return [pltpu.make_async_copy(
                  tok_hbm.at[idx_v.at[b, ti]], rows_v.at[b, ti], gsem[b])
              for ti in range(TC)]

    def store_descs(k, b):
      return [pltpu.make_async_copy(
                  st_v.at[b, :, :, :, pl.ds(0, 128)],
                  out_hbm.at[pl.ds(k * TC, TC), :, wid], ssem[b])]

    idx_desc(0, 0).start()
    idx_desc(0, 0).wait()
    for d in gather_descs(0):
      d.start()
    idx_desc(1, 1).start()

    @pl.loop(0, N_CHUNKS, step=2)
    def _chunks(k0):
      for b in range(2):
        k = k0 + b
        nb = 1 - b

        for d in gather_descs(b):
          d.wait()

        @pl.when(k + 2 < N_CHUNKS)
        def _prefetch_idx():
          idx_desc(k + 2, b).start()

        @pl.when(k + 1 < N_CHUNKS)
        def _issue_next_gather():
          idx_desc(k + 1, nb).wait()
          for d in gather_descs(nb):
            d.start()

        @pl.when(k >= 2)
        def _drain_store():
          for d in store_descs(k - 2, b):
            d.wait()

        for ti in range(TC):
          t = k * TC + ti
          p0 = pos_v[t, pl.ds(0, 16)]
          p1 = pos_v[t, pl.ds(16, 16)]
          st_ti = st_v.at[b, ti]

          @plsc.parallel_loop(0, BW, 8)
          def _rows(bl0):
            i_bl0 = jnp.zeros((16,), jnp.int32) + bl0
            loads = []
            for d in range(8):
              bl = bl0 + d
              loads.append((rows_v[b, ti, bl, pl.ds(0, 16)],
                            rows_v[b, ti, bl, pl.ds(16, 16)]))
            ibls = [i_bl0 + d if d else i_bl0 for d in range(8)]
            sums = [(r0 + p0, r1 + p1) for r0, r1 in loads]
            for d in range(8):
              plsc.store_scatter(st_ti, [i_eg0, i_el, ibls[d]], sums[d][0])
              plsc.store_scatter(st_ti, [i_eg1, i_el, ibls[d]], sums[d][1])

        for d in store_descs(k, b):
          d.start()

    for d in store_descs(N_CHUNKS - 2, 0):
      d.wait()
    for d in store_descs(N_CHUNKS - 1, 1):
      d.wait()

  return emb_kernel


_emb = _build()


def kernel(x, token_table, pos_table):
  xt = x.T.astype(jnp.int32)
  out5 = _emb(token_table, xt, pos_table)
  return out5.transpose(2, 4, 0, 1, 3).reshape(BATCH, MAXLEN, EMBED)


# speedup vs baseline: 1.9899x; 1.0223x over previous
import functools

import jax
import jax.numpy as jnp
from jax import lax
from jax.experimental import pallas as pl
from jax.experimental.pallas import tpu as pltpu
from jax.experimental.pallas import tpu_sc as plsc

VOCAB, MAXLEN, EMBED, BATCH = 100000, 200, 32, 4096
NC, NS = 2, 16
NW = NC * NS
BW = BATCH // NW
TC = 5
N_CHUNKS = MAXLEN // TC
PAD = 133


def _build():
  mesh = plsc.VectorSubcoreMesh(core_axis_name="c", subcore_axis_name="s")

  @functools.partial(
      pl.kernel,
      out_type=jax.ShapeDtypeStruct((MAXLEN, EMBED // 8, BATCH // 128, 8, 128),
                                    jnp.float32),
      mesh=mesh,
      compiler_params=pltpu.CompilerParams(use_tc_tiling_on_sc=False,
                                           needs_layout_passes=False),
      scratch_types=[
          pltpu.VMEM((2, TC, BW), jnp.int32),
          pltpu.VMEM((MAXLEN, EMBED), jnp.float32),
          pltpu.VMEM((2, TC, BW, EMBED), jnp.float32),
          pltpu.VMEM((2, TC, 4, 8, PAD), jnp.float32),
          pltpu.SemaphoreType.DMA,
          pltpu.SemaphoreType.DMA,
          pltpu.SemaphoreType.DMA,
          pltpu.SemaphoreType.DMA,
          pltpu.SemaphoreType.DMA,
          pltpu.SemaphoreType.DMA,
      ],
  )
  def emb_kernel(tok_hbm, xt_hbm, pos_hbm, out_hbm,
                 idx_v, pos_v, rows_v, st_v, g0, g1, s0, s1, i0, i1):
    gsem = (g0, g1)
    ssem = (s0, s1)
    isem = (i0, i1)
    wid = lax.axis_index("s") * NC + lax.axis_index("c")
    bbase = wid * BW

    pltpu.sync_copy(pos_hbm, pos_v)

    lane = lax.iota(jnp.int32, 16)
    i_eg0 = lane // 8
    i_eg1 = i_eg0 + 2
    i_el = lane % 8

    def idx_desc(k, b):
      return pltpu.make_async_copy(
          xt_hbm.at[pl.ds(k * TC, TC), pl.ds(bbase, BW)],
          idx_v.at[b], isem[b])

    def gather_descs(b):
      return [pltpu.make_async_copy(
                  tok_hbm.at[idx_v.at[b, ti]], rows_v.at[b, ti], gsem[b])
              for ti in range(TC)]

    def store_descs(k, b):
      return [pltpu.make_async_copy(
                  st_v.at[b, :, :, :, pl.ds(0, 128)],
                  out_hbm.at[pl.ds(k * TC, TC), :, wid], ssem[b])]

    idx_desc(0, 0).start()
    idx_desc(0, 0).wait()
    for d in gather_descs(0):
      d.start()
    idx_desc(1, 1).start()

    @pl.loop(0, N_CHUNKS, step=2)
    def _chunks(k0):
      for b in range(2):
        k = k0 + b
        nb = 1 - b

        @pl.when(k + 1 < N_CHUNKS)
        def _issue_next_gather():
          idx_desc(k + 1, nb).wait()
          for d in gather_descs(nb):
            d.start()

        for d in gather_descs(b):
          d.wait()

        @pl.when(k + 2 < N_CHUNKS)
        def _prefetch_idx():
          idx_desc(k + 2, b).start()

        @pl.when(k >= 2)
        def _drain_store():
          for d in store_descs(k - 2, b):
            d.wait()

        for ti in range(TC):
          t = k * TC + ti
          p0 = pos_v[t, pl.ds(0, 16)]
          p1 = pos_v[t, pl.ds(16, 16)]
          st_ti = st_v.at[b, ti]

          @plsc.parallel_loop(0, BW, 8)
          def _rows(bl0):
            i_bl0 = jnp.zeros((16,), jnp.int32) + bl0
            loads = []
            for d in range(8):
              bl = bl0 + d
              loads.append((rows_v[b, ti, bl, pl.ds(0, 16)],
                            rows_v[b, ti, bl, pl.ds(16, 16)]))
            ibls = [i_bl0 + d if d else i_bl0 for d in range(8)]
            sums = [(r0 + p0, r1 + p1) for r0, r1 in loads]
            for d in range(8):
              plsc.store_scatter(st_ti, [i_eg0, i_el, ibls[d]], sums[d][0])
              plsc.store_scatter(st_ti, [i_eg1, i_el, ibls[d]], sums[d][1])

        for d in store_descs(k, b):
          d.start()

    for d in store_descs(N_CHUNKS - 2, 0):
      d.wait()
    for d in store_descs(N_CHUNKS - 1, 1):
      d.wait()

  return emb_kernel


_emb = _build()


def kernel(x, token_table, pos_table):
  xt = x.T.astype(jnp.int32)
  out5 = _emb(token_table, xt, pos_table)
  return out5.transpose(2, 4, 0, 1, 3).reshape(BATCH, MAXLEN, EMBED)
